# Initial kernel scaffold; baseline (speedup 1.0000x reference)
import jax
import jax.numpy as jnp
from jax.experimental import pallas as pl


def kernel(feats, edges, W1, b1, W2, b2, W3, b3):
    raise NotImplementedError("write your pallas kernel here")


# speedup vs baseline: 24.3659x; 24.3659x over previous
import functools

import jax
import jax.numpy as jnp
from jax import lax
from jax.experimental import pallas as pl
from jax.experimental.pallas import tpu as pltpu
from jax.experimental.pallas import tpu_sc as plsc

N_NODES = 10000
ROW = 256
NUM_WORKERS = 32
SHARE = 320
ACC_ROWS = 328
TRASH = 320
TOTAL_EDGES = 327680
BLK = 2048
N_BLOCKS = TOTAL_EDGES // BLK
CHUNK = 128
PEND_CAP = 160
KEY_BITS = 15
KEY_MASK = (1 << KEY_BITS) - 1
PAD_KEY = KEY_MASK


def _sc_pool(packed, feats2d):
    mesh = plsc.VectorSubcoreMesh(core_axis_name="c", subcore_axis_name="s")

    @functools.partial(
        pl.kernel,
        out_type=(
            jax.ShapeDtypeStruct((N_NODES, ROW), jnp.float32),
            jax.ShapeDtypeStruct((N_NODES, ROW), jnp.float32),
        ),
        mesh=mesh,
        scratch_types=[
            pltpu.VMEM((ACC_ROWS, ROW), jnp.float32),
            pltpu.VMEM((BLK,), jnp.int32),
            pltpu.VMEM((PEND_CAP,), jnp.int32),
            pltpu.VMEM((PEND_CAP,), jnp.int32),
            pltpu.VMEM((CHUNK, ROW), jnp.float32),
            pltpu.VMEM((32,), jnp.int32),
            pltpu.SemaphoreType.DMA,
        ],
    )
    def k(packed_hbm, feats_hbm, pos_hbm, neg_hbm,
          acc, blk, pend_src, pend_tgt, rowbuf, tmp, sem):
        c = lax.axis_index("c")
        s = lax.axis_index("s")
        w = s * 2 + c
        zeros16 = jnp.zeros((16,), jnp.float32)
        zeros16i = jnp.zeros((16,), jnp.int32)
        trash16 = jnp.full((16,), TRASH, jnp.int32)
        tmp[pl.ds(0, 16)] = zeros16i

        def add_rows(n16):
            def sub(p, carry):
                tgtv = pend_tgt[pl.ds(p * 16, 16)]
                for lane in range(16):
                    t2 = tgtv[lane]
                    r = p * 16 + lane
                    for kk in range(ROW // 16):
                        cs = pl.ds(kk * 16, 16)
                        acc[t2, cs] = acc[t2, cs] + rowbuf[r, cs]
                return carry
            lax.fori_loop(0, n16, sub, 0)

        def fire():
            pltpu.async_copy(
                feats_hbm.at[pend_src.at[pl.ds(0, CHUNK)]],
                rowbuf, sem).wait()
            add_rows(CHUNK // 16)

        for q in range(2):
            lo = q * N_NODES + w * SHARE
            out_hbm = pos_hbm if q == 0 else neg_hbm

            def zero_acc(i, carry):
                acc[i // 16, pl.ds((i % 16) * 16, 16)] = zeros16
                return carry
            lax.fori_loop(0, ACC_ROWS * 16, zero_acc, 0)
            for i in range(PEND_CAP // 16):
                pend_src[pl.ds(i * 16, 16)] = zeros16i
                pend_tgt[pl.ds(i * 16, 16)] = trash16

            def chunk_body(i, off):
                @pl.when(jnp.bitwise_and(i, BLK // 16 - 1) == 0)
                def _():
                    pltpu.sync_copy(
                        packed_hbm.at[pl.ds(
                            lax.shift_right_logical(i, 7) * BLK, BLK)],
                        blk)

                pk = blk[pl.ds(jnp.bitwise_and(i, BLK // 16 - 1) * 16, 16)]
                kv = pk & KEY_MASK
                tv = kv - lo
                m = (tv >= 0) & (tv < SHARE)

                a = jnp.where(m, jnp.full((16,), 1, jnp.int32),
                              jnp.full((16,), 0, jnp.int32))
                for d in (1, 2, 4, 8):
                    tmp[pl.ds(16, 16)] = a
                    a = a + tmp[pl.ds(16 - d, 16)]
                cnt = a[15]

                def hit_path(o):
                    srv = lax.shift_right_logical(pk, KEY_BITS)
                    for l in range(16):
                        t_l = tv[l]
                        ok = (t_l >= 0) & (t_l < SHARE)

                        @pl.when(ok)
                        def _():
                            pend_src[pl.ds(o, 16)] = jnp.full(
                                (16,), srv[l], jnp.int32)
                            pend_tgt[pl.ds(o, 16)] = jnp.full(
                                (16,), t_l, jnp.int32)
                        o = o + jnp.where(ok, 1, 0)
                    return o

                off3 = lax.cond(cnt > 0, hit_path, lambda o: o, off)

                @pl.when(off3 >= CHUNK)
                def _():
                    fire()
                    for z in range(2):
                        pend_src[pl.ds(z * 16, 16)] = \
                            pend_src[pl.ds(CHUNK + z * 16, 16)]
                        pend_tgt[pl.ds(z * 16, 16)] = \
                            pend_tgt[pl.ds(CHUNK + z * 16, 16)]

                return jnp.where(off3 >= CHUNK, off3 - CHUNK, off3)

            off = lax.fori_loop(0, TOTAL_EDGES // 16, chunk_body, 0)

            pend_src[pl.ds(off, 16)] = zeros16i
            pend_tgt[pl.ds(off, 16)] = trash16

            @pl.when(off > 0)
            def _():
                pltpu.async_copy(
                    feats_hbm.at[pend_src.at[pl.ds(0, CHUNK)]],
                    rowbuf, sem).wait()
                add_rows(lax.shift_right_logical(off + 15, 4))

            last = N_NODES - (NUM_WORKERS - 1) * SHARE

            @pl.when(w < NUM_WORKERS - 1)
            def _():
                pltpu.sync_copy(acc.at[pl.ds(0, SHARE)],
                                out_hbm.at[pl.ds(w * SHARE, SHARE)])

            @pl.when(w == NUM_WORKERS - 1)
            def _():
                pltpu.sync_copy(acc.at[pl.ds(0, last)],
                                out_hbm.at[pl.ds(w * SHARE, last)])

    return k(packed, feats2d)


def _conv_matrix(W):
    P = 16
    pi = jnp.arange(P) // 4
    pj = jnp.arange(P) % 4
    a = pi[:, None] - pi[None, :] + 1
    b = pj[:, None] - pj[None, :] + 1
    valid = (a >= 0) & (a <= 2) & (b >= 0) & (b <= 2)
    ac = jnp.clip(a, 0, 2)
    bc = jnp.clip(b, 0, 2)
    Wt = jnp.transpose(W, (1, 0, 2, 3))
    M4 = Wt[:, :, ac, bc] * valid[None, None]
    M = jnp.transpose(M4, (0, 2, 1, 3))
    return M.reshape(W.shape[1] * P, W.shape[0] * P)


def _tc_body(f_ref, p_ref, n_ref, m1a, m1b, m1c, m2, m3, b1r, b2r, b3r, o_ref):
    x = (jnp.dot(f_ref[...], m1a[...], preferred_element_type=jnp.float32)
         + jnp.dot(p_ref[...], m1b[...], preferred_element_type=jnp.float32)
         + jnp.dot(n_ref[...], m1c[...], preferred_element_type=jnp.float32)
         + b1r[...])
    x = jnp.where(x >= 0, x, 0.1 * x)
    x = jnp.dot(x, m2[...], preferred_element_type=jnp.float32) + b2r[...]
    x = jnp.where(x >= 0, x, 0.1 * x)
    x = jnp.dot(x, m3[...], preferred_element_type=jnp.float32) + b3r[...]
    o_ref[...] = jnp.where(x >= 0, x, 0.1 * x)


def _tc_convs(feats2d, pos, neg, M1, M2, M3, b1r, b2r, b3r):
    B = 1000
    full = lambda r, c: pl.BlockSpec((r, c), lambda i: (0, 0))
    return pl.pallas_call(
        _tc_body,
        grid=(N_NODES // B,),
        in_specs=[
            pl.BlockSpec((B, ROW), lambda i: (i, 0)),
            pl.BlockSpec((B, ROW), lambda i: (i, 0)),
            pl.BlockSpec((B, ROW), lambda i: (i, 0)),
            full(ROW, 512), full(ROW, 512), full(ROW, 512),
            full(512, 512), full(512, 256),
            full(1, 512), full(1, 512), full(1, 256),
        ],
        out_specs=pl.BlockSpec((B, ROW), lambda i: (i, 0)),
        out_shape=jax.ShapeDtypeStruct((N_NODES, ROW), jnp.float32),
        compiler_params=pltpu.CompilerParams(
            dimension_semantics=("arbitrary",)),
    )(feats2d, pos, neg, M1[:ROW], M1[ROW:2 * ROW], M1[2 * ROW:], M2, M3,
      b1r.reshape(1, 512), b2r.reshape(1, 512), b3r.reshape(1, 256))


def kernel(feats, edges, W1, b1, W2, b2, W3, b3):
    n = feats.shape[0]
    feats2d = feats.reshape(n, ROW)

    src = edges[:, 0].astype(jnp.int32)
    sign = edges[:, 1]
    dst = edges[:, 2].astype(jnp.int32)
    vsrc = jnp.concatenate([src, dst])
    vdst = jnp.concatenate([dst, src])
    s2 = jnp.concatenate([sign, sign])
    key = vdst + jnp.where(s2 < 0, N_NODES, 0).astype(jnp.int32)
    packed = jnp.left_shift(vsrc, KEY_BITS) | key

    pad = TOTAL_EDGES - packed.shape[0]
    packed = jnp.concatenate(
        [packed, jnp.full((pad,), PAD_KEY, jnp.int32)])

    pos, neg = _sc_pool(packed, feats2d)

    M1 = _conv_matrix(W1)
    M2 = _conv_matrix(W2)
    M3 = _conv_matrix(W3)
    out = _tc_convs(feats2d, pos, neg, M1, M2, M3,
                    jnp.repeat(b1, 16), jnp.repeat(b2, 16), jnp.repeat(b3, 16))
    return out.reshape(n, 16, 4, 4)


# speedup vs baseline: 29.7696x; 1.2218x over previous
import functools

import jax
import jax.numpy as jnp
from jax import lax
from jax.experimental import pallas as pl
from jax.experimental.pallas import tpu as pltpu
from jax.experimental.pallas import tpu_sc as plsc

N_NODES = 10000
ROW = 256
NUM_WORKERS = 32
SHARE = 320
ACC_ROWS = 328
TRASH = 320
TOTAL_EDGES = 327680
BLK = 2048
N_BLOCKS = TOTAL_EDGES // BLK
CHUNK = 128
PEND_CAP = 192
KEY_BITS = 15
KEY_MASK = (1 << KEY_BITS) - 1
PAD_KEY = KEY_MASK


def _sc_pool(packed, feats2d):
    mesh = plsc.VectorSubcoreMesh(core_axis_name="c", subcore_axis_name="s")

    @functools.partial(
        pl.kernel,
        out_type=(
            jax.ShapeDtypeStruct((N_NODES, ROW), jnp.float32),
            jax.ShapeDtypeStruct((N_NODES, ROW), jnp.float32),
        ),
        mesh=mesh,
        scratch_types=[
            pltpu.VMEM((ACC_ROWS, ROW), jnp.float32),
            pltpu.VMEM((BLK,), jnp.int32),
            pltpu.VMEM((PEND_CAP,), jnp.int32),
            pltpu.VMEM((PEND_CAP,), jnp.int32),
            pltpu.VMEM((CHUNK, ROW), jnp.float32),
            pltpu.VMEM((64,), jnp.int32),
            pltpu.SemaphoreType.DMA,
        ],
    )
    def k(packed_hbm, feats_hbm, pos_hbm, neg_hbm,
          acc, blk, pend_src, pend_tgt, rowbuf, tmp, sem):
        c = lax.axis_index("c")
        s = lax.axis_index("s")
        w = s * 2 + c
        zeros16 = jnp.zeros((16,), jnp.float32)
        zeros16i = jnp.zeros((16,), jnp.int32)
        trash16 = jnp.full((16,), TRASH, jnp.int32)
        tmp[pl.ds(0, 16)] = zeros16i
        tmp[pl.ds(32, 16)] = zeros16i

        def add_rows(n16):
            def sub(p, carry):
                tgtv = pend_tgt[pl.ds(p * 16, 16)]
                for lane in range(16):
                    t2 = tgtv[lane]
                    r = p * 16 + lane
                    for kk in range(ROW // 16):
                        cs = pl.ds(kk * 16, 16)
                        acc[t2, cs] = acc[t2, cs] + rowbuf[r, cs]
                return carry
            lax.fori_loop(0, n16, sub, 0)

        def fire():
            pltpu.async_copy(
                feats_hbm.at[pend_src.at[pl.ds(0, CHUNK)]],
                rowbuf, sem).wait()
            add_rows(CHUNK // 16)

        for q in range(2):
            lo = q * N_NODES + w * SHARE
            out_hbm = pos_hbm if q == 0 else neg_hbm

            def zero_acc(i, carry):
                acc[i // 16, pl.ds((i % 16) * 16, 16)] = zeros16
                return carry
            lax.fori_loop(0, ACC_ROWS * 16, zero_acc, 0)
            for i in range(PEND_CAP // 16):
                pend_src[pl.ds(i * 16, 16)] = zeros16i
                pend_tgt[pl.ds(i * 16, 16)] = trash16

            ones16 = jnp.full((16,), 1, jnp.int32)
            zeros16b = jnp.full((16,), 0, jnp.int32)

            def make_hit_path(pk, tv):
                def hit_path(o):
                    srv = lax.shift_right_logical(pk, KEY_BITS)
                    for l in range(16):
                        t_l = tv[l]
                        ok = (t_l >= 0) & (t_l < SHARE)

                        @pl.when(ok)
                        def _():
                            pend_src[pl.ds(o, 16)] = jnp.full(
                                (16,), srv[l], jnp.int32)
                            pend_tgt[pl.ds(o, 16)] = jnp.full(
                                (16,), t_l, jnp.int32)
                        o = o + jnp.where(ok, 1, 0)
                    return o
                return hit_path

            def pair_body(ip, off):
                @pl.when(jnp.bitwise_and(ip, BLK // 32 - 1) == 0)
                def _():
                    pltpu.sync_copy(
                        packed_hbm.at[pl.ds(
                            lax.shift_right_logical(ip, 6) * BLK, BLK)],
                        blk)

                base = jnp.bitwise_and(ip, BLK // 32 - 1) * 32
                pkA = blk[pl.ds(base, 16)]
                pkB = blk[pl.ds(base + 16, 16)]
                kvA = pkA & KEY_MASK
                kvB = pkB & KEY_MASK
                tvA = kvA - lo
                tvB = kvB - lo
                mA = (tvA >= 0) & (tvA < SHARE)
                mB = (tvB >= 0) & (tvB < SHARE)
                a = jnp.where(mA, ones16, zeros16b)
                b = jnp.where(mB, ones16, zeros16b)
                for d in (1, 2, 4, 8):
                    tmp[pl.ds(16, 16)] = a
                    tmp[pl.ds(48, 16)] = b
                    a = a + tmp[pl.ds(16 - d, 16)]
                    b = b + tmp[pl.ds(48 - d, 16)]

                off1 = lax.cond(a[15] > 0, make_hit_path(pkA, tvA),
                                lambda o: o, off)
                off2 = lax.cond(b[15] > 0, make_hit_path(pkB, tvB),
                                lambda o: o, off1)

                @pl.when(off2 >= CHUNK)
                def _():
                    fire()
                    for z in range(2):
                        pend_src[pl.ds(z * 16, 16)] = \
                            pend_src[pl.ds(CHUNK + z * 16, 16)]
                        pend_tgt[pl.ds(z * 16, 16)] = \
                            pend_tgt[pl.ds(CHUNK + z * 16, 16)]

                return jnp.where(off2 >= CHUNK, off2 - CHUNK, off2)

            off = lax.fori_loop(0, TOTAL_EDGES // 32, pair_body, 0)

            pend_src[pl.ds(off, 16)] = zeros16i
            pend_tgt[pl.ds(off, 16)] = trash16

            @pl.when(off > 0)
            def _():
                pltpu.async_copy(
                    feats_hbm.at[pend_src.at[pl.ds(0, CHUNK)]],
                    rowbuf, sem).wait()
                add_rows(lax.shift_right_logical(off + 15, 4))

            last = N_NODES - (NUM_WORKERS - 1) * SHARE

            @pl.when(w < NUM_WORKERS - 1)
            def _():
                pltpu.sync_copy(acc.at[pl.ds(0, SHARE)],
                                out_hbm.at[pl.ds(w * SHARE, SHARE)])

            @pl.when(w == NUM_WORKERS - 1)
            def _():
                pltpu.sync_copy(acc.at[pl.ds(0, last)],
                                out_hbm.at[pl.ds(w * SHARE, last)])

    return k(packed, feats2d)


def _conv_matrix(W):
    P = 16
    pi = jnp.arange(P) // 4
    pj = jnp.arange(P) % 4
    a = pi[:, None] - pi[None, :] + 1
    b = pj[:, None] - pj[None, :] + 1
    valid = (a >= 0) & (a <= 2) & (b >= 0) & (b <= 2)
    ac = jnp.clip(a, 0, 2)
    bc = jnp.clip(b, 0, 2)
    Wt = jnp.transpose(W, (1, 0, 2, 3))
    M4 = Wt[:, :, ac, bc] * valid[None, None]
    M = jnp.transpose(M4, (0, 2, 1, 3))
    return M.reshape(W.shape[1] * P, W.shape[0] * P)


def _tc_body(f_ref, p_ref, n_ref, m1a, m1b, m1c, m2, m3, b1r, b2r, b3r, o_ref):
    x = (jnp.dot(f_ref[...], m1a[...], preferred_element_type=jnp.float32)
         + jnp.dot(p_ref[...], m1b[...], preferred_element_type=jnp.float32)
         + jnp.dot(n_ref[...], m1c[...], preferred_element_type=jnp.float32)
         + b1r[...])
    x = jnp.where(x >= 0, x, 0.1 * x)
    x = jnp.dot(x, m2[...], preferred_element_type=jnp.float32) + b2r[...]
    x = jnp.where(x >= 0, x, 0.1 * x)
    x = jnp.dot(x, m3[...], preferred_element_type=jnp.float32) + b3r[...]
    o_ref[...] = jnp.where(x >= 0, x, 0.1 * x)


def _tc_convs(feats2d, pos, neg, M1, M2, M3, b1r, b2r, b3r):
    B = 1000
    full = lambda r, c: pl.BlockSpec((r, c), lambda i: (0, 0))
    return pl.pallas_call(
        _tc_body,
        grid=(N_NODES // B,),
        in_specs=[
            pl.BlockSpec((B, ROW), lambda i: (i, 0)),
            pl.BlockSpec((B, ROW), lambda i: (i, 0)),
            pl.BlockSpec((B, ROW), lambda i: (i, 0)),
            full(ROW, 512), full(ROW, 512), full(ROW, 512),
            full(512, 512), full(512, 256),
            full(1, 512), full(1, 512), full(1, 256),
        ],
        out_specs=pl.BlockSpec((B, ROW), lambda i: (i, 0)),
        out_shape=jax.ShapeDtypeStruct((N_NODES, ROW), jnp.float32),
        compiler_params=pltpu.CompilerParams(
            dimension_semantics=("arbitrary",)),
    )(feats2d, pos, neg, M1[:ROW], M1[ROW:2 * ROW], M1[2 * ROW:], M2, M3,
      b1r.reshape(1, 512), b2r.reshape(1, 512), b3r.reshape(1, 256))


def kernel(feats, edges, W1, b1, W2, b2, W3, b3):
    n = feats.shape[0]
    feats2d = feats.reshape(n, ROW)

    src = edges[:, 0].astype(jnp.int32)
    sign = edges[:, 1]
    dst = edges[:, 2].astype(jnp.int32)
    vsrc = jnp.concatenate([src, dst])
    vdst = jnp.concatenate([dst, src])
    s2 = jnp.concatenate([sign, sign])
    key = vdst + jnp.where(s2 < 0, N_NODES, 0).astype(jnp.int32)
    packed = jnp.left_shift(vsrc, KEY_BITS) | key

    pad = TOTAL_EDGES - packed.shape[0]
    packed = jnp.concatenate(
        [packed, jnp.full((pad,), PAD_KEY, jnp.int32)])

    pos, neg = _sc_pool(packed, feats2d)

    M1 = _conv_matrix(W1)
    M2 = _conv_matrix(W2)
    M3 = _conv_matrix(W3)
    out = _tc_convs(feats2d, pos, neg, M1, M2, M3,
                    jnp.repeat(b1, 16), jnp.repeat(b2, 16), jnp.repeat(b3, 16))
    return out.reshape(n, 16, 4, 4)


# speedup vs baseline: 34.6528x; 1.1640x over previous
import functools

import jax
import jax.numpy as jnp
from jax import lax
from jax.experimental import pallas as pl
from jax.experimental.pallas import tpu as pltpu
from jax.experimental.pallas import tpu_sc as plsc

N_NODES = 10000
ROW = 256
NUM_WORKERS = 32
SHARE = 320
ACC_ROWS = 328
TRASH = 320
TOTAL_EDGES = 327680
BLK = 2048
N_BLOCKS = TOTAL_EDGES // BLK
CHUNK = 128
PEND_CAP = 192
KEY_BITS = 15
KEY_MASK = (1 << KEY_BITS) - 1
PAD_KEY = KEY_MASK


def _sc_pool(packed, feats2d):
    mesh = plsc.VectorSubcoreMesh(core_axis_name="c", subcore_axis_name="s")

    @functools.partial(
        pl.kernel,
        out_type=(
            jax.ShapeDtypeStruct((N_NODES, ROW), jnp.float32),
            jax.ShapeDtypeStruct((N_NODES, ROW), jnp.float32),
            jax.ShapeDtypeStruct((NUM_WORKERS * TOTAL_EDGES,), jnp.int32),
        ),
        mesh=mesh,
        scratch_types=[
            pltpu.VMEM((ACC_ROWS, ROW), jnp.float32),
            pltpu.VMEM((BLK,), jnp.int32),
            pltpu.VMEM((PEND_CAP,), jnp.int32),
            pltpu.VMEM((PEND_CAP,), jnp.int32),
            pltpu.VMEM((CHUNK,), jnp.int32),
            pltpu.VMEM((CHUNK,), jnp.int32),
            pltpu.VMEM((CHUNK, ROW), jnp.float32),
            pltpu.VMEM((128,), jnp.int32),
            pltpu.SemaphoreType.DMA,
        ],
    )
    def k(packed_hbm, feats_hbm, pos_hbm, neg_hbm, spill_hbm,
          acc, blk, ring0, ring1, gidx, gtgt, rowbuf, tmp, sem):
        c = lax.axis_index("c")
        s = lax.axis_index("s")
        w = s * 2 + c
        zeros16 = jnp.zeros((16,), jnp.float32)
        zeros16i = jnp.zeros((16,), jnp.int32)
        pad16 = jnp.full((16,), TRASH, jnp.int32)
        ones16 = jnp.full((16,), 1, jnp.int32)
        zob = jnp.full((16,), 0, jnp.int32)
        for z in range(4):
            tmp[pl.ds(z * 32, 16)] = zeros16i

        def add_rows(n16):
            def sub(p, carry):
                tgtv = gtgt[pl.ds(p * 16, 16)]
                for lane in range(16):
                    t2 = tgtv[lane]
                    r = p * 16 + lane
                    for kk in range(ROW // 16):
                        cs = pl.ds(kk * 16, 16)
                        acc[t2, cs] = acc[t2, cs] + rowbuf[r, cs]
                return carry
            lax.fori_loop(0, n16, sub, 0)

        def unpack_fire(ring, n16):
            for z in range(CHUNK // 16):
                seg = ring[pl.ds(z * 16, 16)]
                gidx[pl.ds(z * 16, 16)] = lax.shift_right_logical(seg, 9)
                gtgt[pl.ds(z * 16, 16)] = seg & 511
            pltpu.async_copy(feats_hbm.at[gidx], rowbuf, sem).wait()
            add_rows(n16)

        def shift_ring(ring):
            for z in range(2):
                ring[pl.ds(z * 16, 16)] = ring[pl.ds(CHUNK + z * 16, 16)]

        def make_hit_path(ring, pv):
            def hit_path(o):
                for l in range(16):
                    pv_l = pv[l]
                    ok = pv_l >= 0

                    @pl.when(ok)
                    def _():
                        ring[pl.ds(o, 16)] = jnp.full((16,), pv_l, jnp.int32)
                    o = o + jnp.where(ok, 1, 0)
                return o
            return hit_path

        def zero_acc():
            def za(i, carry):
                acc[i // 16, pl.ds((i % 16) * 16, 16)] = zeros16
                return carry
            lax.fori_loop(0, ACC_ROWS * 16, za, 0)

        lo0 = w * SHARE
        lo1 = N_NODES + w * SHARE
        zero_acc()
        for i in range(PEND_CAP // 16):
            ring0[pl.ds(i * 16, 16)] = pad16
            ring1[pl.ds(i * 16, 16)] = pad16

        def pair_body(ip, carry):
            off0, off1, sc = carry

            @pl.when(jnp.bitwise_and(ip, BLK // 32 - 1) == 0)
            def _():
                pltpu.sync_copy(
                    packed_hbm.at[pl.ds(
                        lax.shift_right_logical(ip, 6) * BLK, BLK)],
                    blk)

            base = jnp.bitwise_and(ip, BLK // 32 - 1) * 32
            pkA = blk[pl.ds(base, 16)]
            pkB = blk[pl.ds(base + 16, 16)]
            kvA = pkA & KEY_MASK
            kvB = pkB & KEY_MASK
            srA = lax.shift_right_logical(pkA, KEY_BITS) * 512
            srB = lax.shift_right_logical(pkB, KEY_BITS) * 512
            neg1 = jnp.full((16,), -1, jnp.int32)
            tvs = []
            pvs = []
            for kv, sr in ((kvA, srA), (kvB, srB)):
                for lo in (lo0, lo1):
                    tv = kv - lo
                    m = (tv >= 0) & (tv < SHARE)
                    pvs.append(jnp.where(m, sr | tv, neg1))
                    tvs.append(jnp.where(m, ones16, zob))
            a = list(tvs)
            for d in (1, 2, 4, 8):
                for z in range(4):
                    tmp[pl.ds(z * 32 + 16, 16)] = a[z]
                for z in range(4):
                    a[z] = a[z] + tmp[pl.ds(z * 32 + 16 - d, 16)]

            off0 = lax.cond(a[0][15] > 0, make_hit_path(ring0, pvs[0]),
                            lambda o: o, off0)
            off0 = lax.cond(a[2][15] > 0, make_hit_path(ring0, pvs[2]),
                            lambda o: o, off0)
            off1 = lax.cond(a[1][15] > 0, make_hit_path(ring1, pvs[1]),
                            lambda o: o, off1)
            off1 = lax.cond(a[3][15] > 0, make_hit_path(ring1, pvs[3]),
                            lambda o: o, off1)

            @pl.when(off0 >= CHUNK)
            def _():
                unpack_fire(ring0, CHUNK // 16)
                shift_ring(ring0)

            @pl.when(off1 >= CHUNK)
            def _():
                pltpu.sync_copy(
                    ring1.at[pl.ds(0, CHUNK)],
                    spill_hbm.at[pl.ds(
                        pl.multiple_of(w * TOTAL_EDGES + sc, CHUNK), CHUNK)])
                shift_ring(ring1)

            return (jnp.where(off0 >= CHUNK, off0 - CHUNK, off0),
                    jnp.where(off1 >= CHUNK, off1 - CHUNK, off1),
                    jnp.where(off1 >= CHUNK, sc + CHUNK, sc))

        off0, off1, sc = lax.fori_loop(0, TOTAL_EDGES // 32, pair_body,
                                       (0, 0, 0))

        ring0[pl.ds(off0, 16)] = pad16

        @pl.when(off0 > 0)
        def _():
            unpack_fire(ring0, lax.shift_right_logical(off0 + 15, 4))

        last = N_NODES - (NUM_WORKERS - 1) * SHARE

        @pl.when(w < NUM_WORKERS - 1)
        def _():
            pltpu.sync_copy(acc.at[pl.ds(0, SHARE)],
                            pos_hbm.at[pl.ds(pl.multiple_of(w * SHARE, 8), SHARE)])

        @pl.when(w == NUM_WORKERS - 1)
        def _():
            pltpu.sync_copy(acc.at[pl.ds(0, last)],
                            pos_hbm.at[pl.ds(pl.multiple_of(w * SHARE, 8), last)])

        zero_acc()

        def replay(bkt, carry):
            pltpu.sync_copy(
                spill_hbm.at[pl.ds(
                    pl.multiple_of(w * TOTAL_EDGES + bkt * CHUNK, CHUNK),
                    CHUNK)],
                ring0.at[pl.ds(0, CHUNK)])
            unpack_fire(ring0, CHUNK // 16)
            return carry

        lax.fori_loop(0, lax.shift_right_logical(sc, 7), replay, 0)

        ring1[pl.ds(off1, 16)] = pad16

        @pl.when(off1 > 0)
        def _():
            unpack_fire(ring1, lax.shift_right_logical(off1 + 15, 4))

        @pl.when(w < NUM_WORKERS - 1)
        def _():
            pltpu.sync_copy(acc.at[pl.ds(0, SHARE)],
                            neg_hbm.at[pl.ds(pl.multiple_of(w * SHARE, 8), SHARE)])

        @pl.when(w == NUM_WORKERS - 1)
        def _():
            pltpu.sync_copy(acc.at[pl.ds(0, last)],
                            neg_hbm.at[pl.ds(pl.multiple_of(w * SHARE, 8), last)])

    pos, neg, _ = k(packed, feats2d)
    return pos, neg


def _conv_matrix(W):
    P = 16
    pi = jnp.arange(P) // 4
    pj = jnp.arange(P) % 4
    a = pi[:, None] - pi[None, :] + 1
    b = pj[:, None] - pj[None, :] + 1
    valid = (a >= 0) & (a <= 2) & (b >= 0) & (b <= 2)
    ac = jnp.clip(a, 0, 2)
    bc = jnp.clip(b, 0, 2)
    Wt = jnp.transpose(W, (1, 0, 2, 3))
    M4 = Wt[:, :, ac, bc] * valid[None, None]
    M = jnp.transpose(M4, (0, 2, 1, 3))
    return M.reshape(W.shape[1] * P, W.shape[0] * P)


def _tc_body(f_ref, p_ref, n_ref, m1a, m1b, m1c, m2, m3, b1r, b2r, b3r, o_ref):
    x = (jnp.dot(f_ref[...], m1a[...], preferred_element_type=jnp.float32)
         + jnp.dot(p_ref[...], m1b[...], preferred_element_type=jnp.float32)
         + jnp.dot(n_ref[...], m1c[...], preferred_element_type=jnp.float32)
         + b1r[...])
    x = jnp.where(x >= 0, x, 0.1 * x)
    x = jnp.dot(x, m2[...], preferred_element_type=jnp.float32) + b2r[...]
    x = jnp.where(x >= 0, x, 0.1 * x)
    x = jnp.dot(x, m3[...], preferred_element_type=jnp.float32) + b3r[...]
    o_ref[...] = jnp.where(x >= 0, x, 0.1 * x)


def _tc_convs(feats2d, pos, neg, M1, M2, M3, b1r, b2r, b3r):
    B = 1000
    full = lambda r, c: pl.BlockSpec((r, c), lambda i: (0, 0))
    return pl.pallas_call(
        _tc_body,
        grid=(N_NODES // B,),
        in_specs=[
            pl.BlockSpec((B, ROW), lambda i: (i, 0)),
            pl.BlockSpec((B, ROW), lambda i: (i, 0)),
            pl.BlockSpec((B, ROW), lambda i: (i, 0)),
            full(ROW, 512), full(ROW, 512), full(ROW, 512),
            full(512, 512), full(512, 256),
            full(1, 512), full(1, 512), full(1, 256),
        ],
        out_specs=pl.BlockSpec((B, ROW), lambda i: (i, 0)),
        out_shape=jax.ShapeDtypeStruct((N_NODES, ROW), jnp.float32),
        compiler_params=pltpu.CompilerParams(
            dimension_semantics=("arbitrary",)),
    )(feats2d, pos, neg, M1[:ROW], M1[ROW:2 * ROW], M1[2 * ROW:], M2, M3,
      b1r.reshape(1, 512), b2r.reshape(1, 512), b3r.reshape(1, 256))


def kernel(feats, edges, W1, b1, W2, b2, W3, b3):
    n = feats.shape[0]
    feats2d = feats.reshape(n, ROW)

    src = edges[:, 0].astype(jnp.int32)
    sign = edges[:, 1]
    dst = edges[:, 2].astype(jnp.int32)
    vsrc = jnp.concatenate([src, dst])
    vdst = jnp.concatenate([dst, src])
    s2 = jnp.concatenate([sign, sign])
    key = vdst + jnp.where(s2 < 0, N_NODES, 0).astype(jnp.int32)
    packed = jnp.left_shift(vsrc, KEY_BITS) | key

    pad = TOTAL_EDGES - packed.shape[0]
    packed = jnp.concatenate(
        [packed, jnp.full((pad,), PAD_KEY, jnp.int32)])

    pos, neg = _sc_pool(packed, feats2d)

    M1 = _conv_matrix(W1)
    M2 = _conv_matrix(W2)
    M3 = _conv_matrix(W3)
    out = _tc_convs(feats2d, pos, neg, M1, M2, M3,
                    jnp.repeat(b1, 16), jnp.repeat(b2, 16), jnp.repeat(b3, 16))
    return out.reshape(n, 16, 4, 4)


# speedup vs baseline: 35.6508x; 1.0288x over previous
import functools

import jax
import jax.numpy as jnp
from jax import lax
from jax.experimental import pallas as pl
from jax.experimental.pallas import tpu as pltpu
from jax.experimental.pallas import tpu_sc as plsc

N_NODES = 10000
ROW = 256
NUM_WORKERS = 32
SHARE = 320
ACC_ROWS = 328
TRASH = 320
TOTAL_EDGES = 327680
BLK = 2048
N_BLOCKS = TOTAL_EDGES // BLK
CHUNK = 128
PEND_CAP = 192
KEY_BITS = 15
KEY_MASK = (1 << KEY_BITS) - 1
PAD_KEY = KEY_MASK


def _sc_pool(packed, feats2d):
    mesh = plsc.VectorSubcoreMesh(core_axis_name="c", subcore_axis_name="s")

    @functools.partial(
        pl.kernel,
        out_type=(
            jax.ShapeDtypeStruct((N_NODES, ROW), jnp.float32),
            jax.ShapeDtypeStruct((N_NODES, ROW), jnp.float32),
            jax.ShapeDtypeStruct((NUM_WORKERS * TOTAL_EDGES,), jnp.int32),
        ),
        mesh=mesh,
        scratch_types=[
            pltpu.VMEM((ACC_ROWS, ROW), jnp.float32),
            pltpu.VMEM((2, BLK), jnp.int32),
            pltpu.VMEM((PEND_CAP,), jnp.int32),
            pltpu.VMEM((PEND_CAP,), jnp.int32),
            pltpu.VMEM((CHUNK,), jnp.int32),
            pltpu.VMEM((CHUNK,), jnp.int32),
            pltpu.VMEM((CHUNK, ROW), jnp.float32),
            pltpu.VMEM((128,), jnp.int32),
            pltpu.SemaphoreType.DMA,
            pltpu.SemaphoreType.DMA,
        ],
    )
    def k(packed_hbm, feats_hbm, pos_hbm, neg_hbm, spill_hbm,
          acc, blk, ring0, ring1, gidx, gtgt, rowbuf, tmp, sem, msem):
        c = lax.axis_index("c")
        s = lax.axis_index("s")
        w = s * 2 + c
        zeros16 = jnp.zeros((16,), jnp.float32)
        zeros16i = jnp.zeros((16,), jnp.int32)
        pad16 = jnp.full((16,), TRASH, jnp.int32)
        ones16 = jnp.full((16,), 1, jnp.int32)
        zob = jnp.full((16,), 0, jnp.int32)
        for z in range(4):
            tmp[pl.ds(z * 32, 16)] = zeros16i

        def add_rows(n16):
            def sub(p, carry):
                tgtv = gtgt[pl.ds(p * 16, 16)]
                for lane in range(16):
                    t2 = tgtv[lane]
                    r = p * 16 + lane
                    for kk in range(ROW // 16):
                        cs = pl.ds(kk * 16, 16)
                        acc[t2, cs] = acc[t2, cs] + rowbuf[r, cs]
                return carry
            lax.fori_loop(0, n16, sub, 0)

        def unpack_fire(ring, n16):
            for z in range(CHUNK // 16):
                seg = ring[pl.ds(z * 16, 16)]
                gidx[pl.ds(z * 16, 16)] = lax.shift_right_logical(seg, 9)
                gtgt[pl.ds(z * 16, 16)] = seg & 511
            pltpu.async_copy(feats_hbm.at[gidx], rowbuf, sem).wait()
            add_rows(n16)

        def shift_ring(ring):
            for z in range(2):
                ring[pl.ds(z * 16, 16)] = ring[pl.ds(CHUNK + z * 16, 16)]

        def make_hit_path(ring, pv):
            def hit_path(o):
                for l in range(16):
                    pv_l = pv[l]
                    ok = pv_l >= 0

                    @pl.when(ok)
                    def _():
                        ring[pl.ds(o, 16)] = jnp.full((16,), pv_l, jnp.int32)
                    o = o + jnp.where(ok, 1, 0)
                return o
            return hit_path

        def zero_acc():
            def za(i, carry):
                acc[i // 16, pl.ds((i % 16) * 16, 16)] = zeros16
                return carry
            lax.fori_loop(0, ACC_ROWS * 16, za, 0)

        lo0 = w * SHARE
        lo1 = N_NODES + w * SHARE
        zero_acc()
        for i in range(PEND_CAP // 16):
            ring0[pl.ds(i * 16, 16)] = pad16
            ring1[pl.ds(i * 16, 16)] = pad16

        pltpu.async_copy(packed_hbm.at[pl.ds(0, BLK)], blk.at[0],
                         msem).wait()
        pltpu.async_copy(packed_hbm.at[pl.ds(BLK, BLK)], blk.at[1], msem)

        def pair_body(ip, carry):
            off0, off1, sc = carry
            bsel = jnp.bitwise_and(lax.shift_right_logical(ip, 6), 1)

            @pl.when((jnp.bitwise_and(ip, BLK // 32 - 1) == 0) & (ip > 0))
            def _():
                pltpu.make_async_copy(
                    packed_hbm.at[pl.ds(0, BLK)], blk.at[bsel], msem).wait()
                pltpu.async_copy(
                    packed_hbm.at[pl.ds(
                        (lax.shift_right_logical(ip, 6) + 1) * BLK, BLK)],
                    blk.at[1 - bsel], msem)

            base = jnp.bitwise_and(ip, BLK // 32 - 1) * 32
            pkA = blk[bsel, pl.ds(base, 16)]
            pkB = blk[bsel, pl.ds(base + 16, 16)]
            kvA = pkA & KEY_MASK
            kvB = pkB & KEY_MASK
            srA = lax.shift_right_logical(pkA, KEY_BITS) * 512
            srB = lax.shift_right_logical(pkB, KEY_BITS) * 512
            neg1 = jnp.full((16,), -1, jnp.int32)
            tvs = []
            pvs = []
            for kv, sr in ((kvA, srA), (kvB, srB)):
                for lo in (lo0, lo1):
                    tv = kv - lo
                    m = (tv >= 0) & (tv < SHARE)
                    pvs.append(jnp.where(m, sr | tv, neg1))
                    tvs.append(jnp.where(m, ones16, zob))
            a = list(tvs)
            for d in (1, 2, 4, 8):
                for z in range(4):
                    tmp[pl.ds(z * 32 + 16, 16)] = a[z]
                for z in range(4):
                    a[z] = a[z] + tmp[pl.ds(z * 32 + 16 - d, 16)]

            off0 = lax.cond(a[0][15] > 0, make_hit_path(ring0, pvs[0]),
                            lambda o: o, off0)
            off0 = lax.cond(a[2][15] > 0, make_hit_path(ring0, pvs[2]),
                            lambda o: o, off0)
            off1 = lax.cond(a[1][15] > 0, make_hit_path(ring1, pvs[1]),
                            lambda o: o, off1)
            off1 = lax.cond(a[3][15] > 0, make_hit_path(ring1, pvs[3]),
                            lambda o: o, off1)

            @pl.when(off0 >= CHUNK)
            def _():
                unpack_fire(ring0, CHUNK // 16)
                shift_ring(ring0)

            @pl.when(off1 >= CHUNK)
            def _():
                pltpu.sync_copy(
                    ring1.at[pl.ds(0, CHUNK)],
                    spill_hbm.at[pl.ds(
                        pl.multiple_of(w * TOTAL_EDGES + sc, CHUNK), CHUNK)])
                shift_ring(ring1)

            return (jnp.where(off0 >= CHUNK, off0 - CHUNK, off0),
                    jnp.where(off1 >= CHUNK, off1 - CHUNK, off1),
                    jnp.where(off1 >= CHUNK, sc + CHUNK, sc))

        off0, off1, sc = lax.fori_loop(0, TOTAL_EDGES // 32, pair_body,
                                       (0, 0, 0))
        pltpu.make_async_copy(packed_hbm.at[pl.ds(0, BLK)], blk.at[0],
                              msem).wait()

        ring0[pl.ds(off0, 16)] = pad16

        @pl.when(off0 > 0)
        def _():
            unpack_fire(ring0, lax.shift_right_logical(off0 + 15, 4))

        last = N_NODES - (NUM_WORKERS - 1) * SHARE

        @pl.when(w < NUM_WORKERS - 1)
        def _():
            pltpu.sync_copy(acc.at[pl.ds(0, SHARE)],
                            pos_hbm.at[pl.ds(pl.multiple_of(w * SHARE, 8), SHARE)])

        @pl.when(w == NUM_WORKERS - 1)
        def _():
            pltpu.sync_copy(acc.at[pl.ds(0, last)],
                            pos_hbm.at[pl.ds(pl.multiple_of(w * SHARE, 8), last)])

        zero_acc()

        def replay(bkt, carry):
            pltpu.sync_copy(
                spill_hbm.at[pl.ds(
                    pl.multiple_of(w * TOTAL_EDGES + bkt * CHUNK, CHUNK),
                    CHUNK)],
                ring0.at[pl.ds(0, CHUNK)])
            unpack_fire(ring0, CHUNK // 16)
            return carry

        lax.fori_loop(0, lax.shift_right_logical(sc, 7), replay, 0)

        ring1[pl.ds(off1, 16)] = pad16

        @pl.when(off1 > 0)
        def _():
            unpack_fire(ring1, lax.shift_right_logical(off1 + 15, 4))

        @pl.when(w < NUM_WORKERS - 1)
        def _():
            pltpu.sync_copy(acc.at[pl.ds(0, SHARE)],
                            neg_hbm.at[pl.ds(pl.multiple_of(w * SHARE, 8), SHARE)])

        @pl.when(w == NUM_WORKERS - 1)
        def _():
            pltpu.sync_copy(acc.at[pl.ds(0, last)],
                            neg_hbm.at[pl.ds(pl.multiple_of(w * SHARE, 8), last)])

    pos, neg, _ = k(packed, feats2d)
    return pos, neg


def _conv_matrix(W):
    P = 16
    pi = jnp.arange(P) // 4
    pj = jnp.arange(P) % 4
    a = pi[:, None] - pi[None, :] + 1
    b = pj[:, None] - pj[None, :] + 1
    valid = (a >= 0) & (a <= 2) & (b >= 0) & (b <= 2)
    ac = jnp.clip(a, 0, 2)
    bc = jnp.clip(b, 0, 2)
    Wt = jnp.transpose(W, (1, 0, 2, 3))
    M4 = Wt[:, :, ac, bc] * valid[None, None]
    M = jnp.transpose(M4, (0, 2, 1, 3))
    return M.reshape(W.shape[1] * P, W.shape[0] * P)


def _tc_body(f_ref, p_ref, n_ref, m1a, m1b, m1c, m2, m3, b1r, b2r, b3r, o_ref):
    x = (jnp.dot(f_ref[...], m1a[...], preferred_element_type=jnp.float32)
         + jnp.dot(p_ref[...], m1b[...], preferred_element_type=jnp.float32)
         + jnp.dot(n_ref[...], m1c[...], preferred_element_type=jnp.float32)
         + b1r[...])
    x = jnp.where(x >= 0, x, 0.1 * x)
    x = jnp.dot(x, m2[...], preferred_element_type=jnp.float32) + b2r[...]
    x = jnp.where(x >= 0, x, 0.1 * x)
    x = jnp.dot(x, m3[...], preferred_element_type=jnp.float32) + b3r[...]
    o_ref[...] = jnp.where(x >= 0, x, 0.1 * x)


def _tc_convs(feats2d, pos, neg, M1, M2, M3, b1r, b2r, b3r):
    B = 1000
    full = lambda r, c: pl.BlockSpec((r, c), lambda i: (0, 0))
    return pl.pallas_call(
        _tc_body,
        grid=(N_NODES // B,),
        in_specs=[
            pl.BlockSpec((B, ROW), lambda i: (i, 0)),
            pl.BlockSpec((B, ROW), lambda i: (i, 0)),
            pl.BlockSpec((B, ROW), lambda i: (i, 0)),
            full(ROW, 512), full(ROW, 512), full(ROW, 512),
            full(512, 512), full(512, 256),
            full(1, 512), full(1, 512), full(1, 256),
        ],
        out_specs=pl.BlockSpec((B, ROW), lambda i: (i, 0)),
        out_shape=jax.ShapeDtypeStruct((N_NODES, ROW), jnp.float32),
        compiler_params=pltpu.CompilerParams(
            dimension_semantics=("arbitrary",)),
    )(feats2d, pos, neg, M1[:ROW], M1[ROW:2 * ROW], M1[2 * ROW:], M2, M3,
      b1r.reshape(1, 512), b2r.reshape(1, 512), b3r.reshape(1, 256))


def kernel(feats, edges, W1, b1, W2, b2, W3, b3):
    n = feats.shape[0]
    feats2d = feats.reshape(n, ROW)

    src = edges[:, 0].astype(jnp.int32)
    sign = edges[:, 1]
    dst = edges[:, 2].astype(jnp.int32)
    vsrc = jnp.concatenate([src, dst])
    vdst = jnp.concatenate([dst, src])
    s2 = jnp.concatenate([sign, sign])
    key = vdst + jnp.where(s2 < 0, N_NODES, 0).astype(jnp.int32)
    packed = jnp.left_shift(vsrc, KEY_BITS) | key

    pad = TOTAL_EDGES + BLK - packed.shape[0]
    packed = jnp.concatenate(
        [packed, jnp.full((pad,), PAD_KEY, jnp.int32)])

    pos, neg = _sc_pool(packed, feats2d)

    M1 = _conv_matrix(W1)
    M2 = _conv_matrix(W2)
    M3 = _conv_matrix(W3)
    out = _tc_convs(feats2d, pos, neg, M1, M2, M3,
                    jnp.repeat(b1, 16), jnp.repeat(b2, 16), jnp.repeat(b3, 16))
    return out.reshape(n, 16, 4, 4)


# speedup vs baseline: 35.7959x; 1.0041x over previous
import functools

import jax
import jax.numpy as jnp
from jax import lax
from jax.experimental import pallas as pl
from jax.experimental.pallas import tpu as pltpu
from jax.experimental.pallas import tpu_sc as plsc

N_NODES = 10000
ROW = 256
NUM_WORKERS = 32
SHARE = 320
ACC_ROWS = 328
TRASH = 320
TOTAL_EDGES = 327680
BLK = 2048
N_BLOCKS = TOTAL_EDGES // BLK
CHUNK = 128
PEND_CAP = 192
KEY_BITS = 15
KEY_MASK = (1 << KEY_BITS) - 1
PAD_KEY = KEY_MASK


def _sc_pool(packed, feats2d):
    mesh = plsc.VectorSubcoreMesh(core_axis_name="c", subcore_axis_name="s")

    @functools.partial(
        pl.kernel,
        out_type=(
            jax.ShapeDtypeStruct((N_NODES, ROW), jnp.float32),
            jax.ShapeDtypeStruct((N_NODES, ROW), jnp.float32),
            jax.ShapeDtypeStruct((NUM_WORKERS * TOTAL_EDGES,), jnp.int32),
        ),
        mesh=mesh,
        scratch_types=[
            pltpu.VMEM((ACC_ROWS, ROW), jnp.float32),
            pltpu.VMEM((2, BLK), jnp.int32),
            pltpu.VMEM((PEND_CAP,), jnp.int32),
            pltpu.VMEM((PEND_CAP,), jnp.int32),
            pltpu.VMEM((CHUNK,), jnp.int32),
            pltpu.VMEM((CHUNK,), jnp.int32),
            pltpu.VMEM((CHUNK, ROW), jnp.float32),
            pltpu.VMEM((32,), jnp.int32),
            pltpu.VMEM((32,), jnp.int32),
            pltpu.SemaphoreType.DMA,
            pltpu.SemaphoreType.DMA,
        ],
    )
    def k(packed_hbm, feats_hbm, pos_hbm, neg_hbm, spill_hbm,
          acc, blk, ring0, ring1, gidx, gtgt, rowbuf, tmp, tmp2, sem, msem):
        c = lax.axis_index("c")
        s = lax.axis_index("s")
        w = s * 2 + c
        zeros16 = jnp.zeros((16,), jnp.float32)
        zeros16i = jnp.zeros((16,), jnp.int32)
        pad16 = jnp.full((16,), TRASH, jnp.int32)
        ones16 = jnp.full((16,), 1, jnp.int32)
        zob = jnp.full((16,), 0, jnp.int32)
        tmp[pl.ds(0, 16)] = zeros16i
        tmp2[pl.ds(0, 16)] = zeros16i

        def add_rows(n16):
            def sub(p, carry):
                tgtv = gtgt[pl.ds(p * 16, 16)]
                for lane in range(16):
                    t2 = tgtv[lane]
                    r = p * 16 + lane
                    for kk in range(ROW // 16):
                        cs = pl.ds(kk * 16, 16)
                        acc[t2, cs] = acc[t2, cs] + rowbuf[r, cs]
                return carry
            lax.fori_loop(0, n16, sub, 0)

        def unpack_fire(ring, n16):
            for z in range(CHUNK // 16):
                seg = ring[pl.ds(z * 16, 16)]
                gidx[pl.ds(z * 16, 16)] = lax.shift_right_logical(seg, 10)
                gtgt[pl.ds(z * 16, 16)] = seg & 511
            pltpu.async_copy(feats_hbm.at[gidx], rowbuf, sem).wait()
            add_rows(n16)

        def shift_ring(ring):
            for z in range(2):
                ring[pl.ds(z * 16, 16)] = ring[pl.ds(CHUNK + z * 16, 16)]

        def make_hit_path(pv):
            def hit_path(oo):
                o0, o1 = oo
                for l in range(16):
                    pv_l = pv[l]
                    ok = pv_l >= 0
                    is1 = jnp.bitwise_and(pv_l, 512) != 0
                    ok0 = ok & jnp.logical_not(is1)
                    ok1 = ok & is1

                    @pl.when(ok0)
                    def _():
                        ring0[pl.ds(o0, 16)] = jnp.full((16,), pv_l,
                                                        jnp.int32)

                    @pl.when(ok1)
                    def _():
                        ring1[pl.ds(o1, 16)] = jnp.full((16,), pv_l,
                                                        jnp.int32)
                    o0 = o0 + jnp.where(ok0, 1, 0)
                    o1 = o1 + jnp.where(ok1, 1, 0)
                return (o0, o1)
            return hit_path

        def zero_acc():
            def za(i, carry):
                acc[i // 16, pl.ds((i % 16) * 16, 16)] = zeros16
                return carry
            lax.fori_loop(0, ACC_ROWS * 16, za, 0)

        lo0 = w * SHARE
        lo1 = N_NODES + w * SHARE
        zero_acc()
        for i in range(PEND_CAP // 16):
            ring0[pl.ds(i * 16, 16)] = pad16
            ring1[pl.ds(i * 16, 16)] = pad16

        pltpu.async_copy(packed_hbm.at[pl.ds(0, BLK)], blk.at[0],
                         msem).wait()
        pltpu.async_copy(packed_hbm.at[pl.ds(BLK, BLK)], blk.at[1], msem)

        def pair_body(ip, carry):
            off0, off1, sc = carry
            bsel = jnp.bitwise_and(lax.shift_right_logical(ip, 6), 1)

            @pl.when((jnp.bitwise_and(ip, BLK // 32 - 1) == 0) & (ip > 0))
            def _():
                pltpu.make_async_copy(
                    packed_hbm.at[pl.ds(0, BLK)], blk.at[bsel], msem).wait()
                pltpu.async_copy(
                    packed_hbm.at[pl.ds(
                        (lax.shift_right_logical(ip, 6) + 1) * BLK, BLK)],
                    blk.at[1 - bsel], msem)

            base = jnp.bitwise_and(ip, BLK // 32 - 1) * 32
            pkA = blk[bsel, pl.ds(base, 16)]
            pkB = blk[bsel, pl.ds(base + 16, 16)]
            neg1 = jnp.full((16,), -1, jnp.int32)
            pvs = []
            gs = []
            for pk in (pkA, pkB):
                kv = pk & KEY_MASK
                sr = lax.shift_right_logical(pk, KEY_BITS) * 1024
                tv0 = kv - lo0
                tv1 = kv - lo1
                m0 = (tv0 >= 0) & (tv0 < SHARE)
                m1 = (tv1 >= 0) & (tv1 < SHARE)
                pv = jnp.where(m0, sr | tv0,
                               jnp.where(m1, (sr | 512) | tv1, neg1))
                pvs.append(pv)
                gs.append(jnp.where(m0 | m1, ones16, zob))
            a0, a1 = gs
            for d in (1, 2, 4, 8):
                tmp[pl.ds(16, 16)] = a0
                tmp2[pl.ds(16, 16)] = a1
                a0 = a0 + tmp[pl.ds(16 - d, 16)]
                a1 = a1 + tmp2[pl.ds(16 - d, 16)]

            off0, off1 = lax.cond(a0[15] > 0, make_hit_path(pvs[0]),
                                  lambda oo: oo, (off0, off1))
            off0, off1 = lax.cond(a1[15] > 0, make_hit_path(pvs[1]),
                                  lambda oo: oo, (off0, off1))

            @pl.when(off0 >= CHUNK)
            def _():
                unpack_fire(ring0, CHUNK // 16)
                shift_ring(ring0)

            @pl.when(off1 >= CHUNK)
            def _():
                pltpu.sync_copy(
                    ring1.at[pl.ds(0, CHUNK)],
                    spill_hbm.at[pl.ds(
                        pl.multiple_of(w * TOTAL_EDGES + sc, CHUNK), CHUNK)])
                shift_ring(ring1)

            return (jnp.where(off0 >= CHUNK, off0 - CHUNK, off0),
                    jnp.where(off1 >= CHUNK, off1 - CHUNK, off1),
                    jnp.where(off1 >= CHUNK, sc + CHUNK, sc))

        off0, off1, sc = lax.fori_loop(0, TOTAL_EDGES // 32, pair_body,
                                       (0, 0, 0))
        pltpu.make_async_copy(packed_hbm.at[pl.ds(0, BLK)], blk.at[0],
                              msem).wait()

        ring0[pl.ds(off0, 16)] = pad16

        @pl.when(off0 > 0)
        def _():
            unpack_fire(ring0, lax.shift_right_logical(off0 + 15, 4))

        last = N_NODES - (NUM_WORKERS - 1) * SHARE

        @pl.when(w < NUM_WORKERS - 1)
        def _():
            pltpu.sync_copy(acc.at[pl.ds(0, SHARE)],
                            pos_hbm.at[pl.ds(pl.multiple_of(w * SHARE, 8), SHARE)])

        @pl.when(w == NUM_WORKERS - 1)
        def _():
            pltpu.sync_copy(acc.at[pl.ds(0, last)],
                            pos_hbm.at[pl.ds(pl.multiple_of(w * SHARE, 8), last)])

        zero_acc()

        def replay(bkt, carry):
            pltpu.sync_copy(
                spill_hbm.at[pl.ds(
                    pl.multiple_of(w * TOTAL_EDGES + bkt * CHUNK, CHUNK),
                    CHUNK)],
                ring0.at[pl.ds(0, CHUNK)])
            unpack_fire(ring0, CHUNK // 16)
            return carry

        lax.fori_loop(0, lax.shift_right_logical(sc, 7), replay, 0)

        ring1[pl.ds(off1, 16)] = pad16

        @pl.when(off1 > 0)
        def _():
            unpack_fire(ring1, lax.shift_right_logical(off1 + 15, 4))

        @pl.when(w < NUM_WORKERS - 1)
        def _():
            pltpu.sync_copy(acc.at[pl.ds(0, SHARE)],
                            neg_hbm.at[pl.ds(pl.multiple_of(w * SHARE, 8), SHARE)])

        @pl.when(w == NUM_WORKERS - 1)
        def _():
            pltpu.sync_copy(acc.at[pl.ds(0, last)],
                            neg_hbm.at[pl.ds(pl.multiple_of(w * SHARE, 8), last)])

    pos, neg, _ = k(packed, feats2d)
    return pos, neg


def _conv_matrix(W):
    P = 16
    pi = jnp.arange(P) // 4
    pj = jnp.arange(P) % 4
    a = pi[:, None] - pi[None, :] + 1
    b = pj[:, None] - pj[None, :] + 1
    valid = (a >= 0) & (a <= 2) & (b >= 0) & (b <= 2)
    ac = jnp.clip(a, 0, 2)
    bc = jnp.clip(b, 0, 2)
    Wt = jnp.transpose(W, (1, 0, 2, 3))
    M4 = Wt[:, :, ac, bc] * valid[None, None]
    M = jnp.transpose(M4, (0, 2, 1, 3))
    return M.reshape(W.shape[1] * P, W.shape[0] * P)


def _tc_body(f_ref, p_ref, n_ref, m1a, m1b, m1c, m2, m3, b1r, b2r, b3r, o_ref):
    x = (jnp.dot(f_ref[...], m1a[...], preferred_element_type=jnp.float32)
         + jnp.dot(p_ref[...], m1b[...], preferred_element_type=jnp.float32)
         + jnp.dot(n_ref[...], m1c[...], preferred_element_type=jnp.float32)
         + b1r[...])
    x = jnp.where(x >= 0, x, 0.1 * x)
    x = jnp.dot(x, m2[...], preferred_element_type=jnp.float32) + b2r[...]
    x = jnp.where(x >= 0, x, 0.1 * x)
    x = jnp.dot(x, m3[...], preferred_element_type=jnp.float32) + b3r[...]
    o_ref[...] = jnp.where(x >= 0, x, 0.1 * x)


def _tc_convs(feats2d, pos, neg, M1, M2, M3, b1r, b2r, b3r):
    B = 1000
    full = lambda r, c: pl.BlockSpec((r, c), lambda i: (0, 0))
    return pl.pallas_call(
        _tc_body,
        grid=(N_NODES // B,),
        in_specs=[
            pl.BlockSpec((B, ROW), lambda i: (i, 0)),
            pl.BlockSpec((B, ROW), lambda i: (i, 0)),
            pl.BlockSpec((B, ROW), lambda i: (i, 0)),
            full(ROW, 512), full(ROW, 512), full(ROW, 512),
            full(512, 512), full(512, 256),
            full(1, 512), full(1, 512), full(1, 256),
        ],
        out_specs=pl.BlockSpec((B, ROW), lambda i: (i, 0)),
        out_shape=jax.ShapeDtypeStruct((N_NODES, ROW), jnp.float32),
        compiler_params=pltpu.CompilerParams(
            dimension_semantics=("arbitrary",)),
    )(feats2d, pos, neg, M1[:ROW], M1[ROW:2 * ROW], M1[2 * ROW:], M2, M3,
      b1r.reshape(1, 512), b2r.reshape(1, 512), b3r.reshape(1, 256))


def kernel(feats, edges, W1, b1, W2, b2, W3, b3):
    n = feats.shape[0]
    feats2d = feats.reshape(n, ROW)

    src = edges[:, 0].astype(jnp.int32)
    sign = edges[:, 1]
    dst = edges[:, 2].astype(jnp.int32)
    vsrc = jnp.concatenate([src, dst])
    vdst = jnp.concatenate([dst, src])
    s2 = jnp.concatenate([sign, sign])
    key = vdst + jnp.where(s2 < 0, N_NODES, 0).astype(jnp.int32)
    packed = jnp.left_shift(vsrc, KEY_BITS) | key

    pad = TOTAL_EDGES + BLK - packed.shape[0]
    packed = jnp.concatenate(
        [packed, jnp.full((pad,), PAD_KEY, jnp.int32)])

    pos, neg = _sc_pool(packed, feats2d)

    M1 = _conv_matrix(W1)
    M2 = _conv_matrix(W2)
    M3 = _conv_matrix(W3)
    out = _tc_convs(feats2d, pos, neg, M1, M2, M3,
                    jnp.repeat(b1, 16), jnp.repeat(b2, 16), jnp.repeat(b3, 16))
    return out.reshape(n, 16, 4, 4)
